# baseline (speedup 1.0000x reference)
import jax
import jax.numpy as jnp
from jax.experimental import pallas as pl
from jax.experimental.pallas import tpu as pltpu

_NX, _NY, _NZ = 432, 496, 1
_C = 64
_P = 100
_ROWS = 16
_NBLK = _NY // _ROWS


def _scatter_kernel(flags_ref, coords_ref, feats_ref, out_ref):
    b = pl.program_id(0)

    @pl.when(flags_ref[b] == 0)
    def _zero():
        out_ref[...] = jnp.zeros_like(out_ref)

    @pl.when(flags_ref[b] != 0)
    def _scatter():
        coords = coords_ref[...]
        idx = coords[:, 1:2] + coords[:, 2:3] * _NX + coords[:, 3:4]
        feats = feats_ref[...]
        for r in range(_ROWS):
            y = b * _ROWS + r
            cols = jax.lax.broadcasted_iota(jnp.int32, (_P, _NX), 1) + y * _NX
            onehot = (idx == cols).astype(jnp.float32)
            row = jax.lax.dot_general(
                feats, onehot, (((0,), (0,)), ((), ())),
                preferred_element_type=jnp.float32)
            out_ref[0, :, r, :] = row


def kernel(pillar_features, voxel_coords):
    coords = voxel_coords.astype(jnp.int32)
    indices = coords[:, 1] + coords[:, 2] * _NX + coords[:, 3]
    flags = jnp.zeros((_NBLK,), jnp.int32).at[indices // (_NX * _ROWS)].set(
        1, mode="drop")

    grid_spec = pltpu.PrefetchScalarGridSpec(
        num_scalar_prefetch=1,
        grid=(_NBLK,),
        in_specs=[
            pl.BlockSpec((_P, 4), lambda b, flags: (0, 0)),
            pl.BlockSpec((_P, _C), lambda b, flags: (0, 0)),
        ],
        out_specs=pl.BlockSpec((1, _C, _ROWS, _NX), lambda b, flags: (0, 0, b, 0)),
    )
    out = pl.pallas_call(
        _scatter_kernel,
        grid_spec=grid_spec,
        out_shape=jax.ShapeDtypeStruct((1, _C * _NZ, _NY, _NX), jnp.float32),
        compiler_params=pltpu.CompilerParams(
            dimension_semantics=("parallel",)),
    )(flags, coords, pillar_features[:_P, :])
    return out
